# Initial kernel scaffold; baseline (speedup 1.0000x reference)
import jax
import jax.numpy as jnp
from jax.experimental import pallas as pl


def kernel(value, depth, position, value_table, spatial_table_0, spatial_table_1, spatial_table_2, eos):
    raise NotImplementedError("write your pallas kernel here")


# speedup vs baseline: 3.9330x; 3.9330x over previous
import functools

import jax
import jax.numpy as jnp
from jax import lax
from jax.experimental import pallas as pl
from jax.experimental.pallas import tpu as pltpu
from jax.experimental.pallas import tpu_sc as plsc

EMBED = 64
SEQ = 200
HALF = 100
NLANE = 16
NREG = EMBED // NLANE


def _sc_kernel(n_seqs, seq_per_worker):
  mesh = plsc.VectorSubcoreMesh(
      core_axis_name="c", subcore_axis_name="s", num_cores=2, num_subcores=16)

  @functools.partial(
      pl.kernel,
      out_type=jax.ShapeDtypeStruct((n_seqs * SEQ, EMBED), jnp.float32),
      mesh=mesh,
      compiler_params=pltpu.CompilerParams(use_tc_tiling_on_sc=False),
      scratch_types=[
          [pltpu.VMEM((HALF,), jnp.int32) for _ in range(2)],
          [pltpu.VMEM((HALF,), jnp.int32) for _ in range(6)],
          pltpu.VMEM((SEQ, EMBED), jnp.float32),
          pltpu.VMEM((3 * SEQ, EMBED), jnp.float32),
          pltpu.VMEM((SEQ, EMBED), jnp.float32),
          pltpu.VMEM((EMBED,), jnp.float32),
          pltpu.SemaphoreType.DMA,
      ],
  )
  def body(val_hbm, pos_hbm, vtab_hbm, stab_hbm, eos_hbm, out_hbm,
           vidx, pidx, vrows, srows, outb, eosb, sem):
    cid = lax.axis_index("c")
    sid = lax.axis_index("s")
    wid = sid * 2 + cid

    pltpu.sync_copy(eos_hbm, eosb)
    eosv = [eosb[pl.ds(e * NLANE, NLANE)] for e in range(NREG)]

    def seq_body(i, _):
      seq = wid * seq_per_worker + i
      for h in range(2):
        pltpu.sync_copy(val_hbm.at[seq * 2 + h], vidx[h])
      for h in range(6):
        pltpu.sync_copy(pos_hbm.at[seq * 6 + h], pidx[h])
      copies = []
      for h in range(2):
        copies.append(pltpu.async_copy(
            vtab_hbm.at[vidx[h]],
            vrows.at[pl.ds(h * HALF, HALF)], sem))
      for h in range(6):
        copies.append(pltpu.async_copy(
            stab_hbm.at[pidx[h]],
            srows.at[pl.ds(h * HALF, HALF)], sem))
      for cp in copies:
        cp.wait()

      pe0 = tuple(
          srows[0, pl.ds(e * NLANE, NLANE)]
          + srows[1, pl.ds(e * NLANE, NLANE)]
          + srows[2, pl.ds(e * NLANE, NLANE)]
          for e in range(NREG))

      def tok_body(s, pe):
        nxt = []
        for e in range(NREG):
          d = pl.ds(e * NLANE, NLANE)
          pn = (srows[3 * s + 3, d] + srows[3 * s + 4, d]
                + srows[3 * s + 5, d])
          outb[s, d] = vrows[s, d] + pe[e] + pn
          nxt.append(pn)
        return tuple(nxt)

      pe_last = lax.fori_loop(0, SEQ - 1, tok_body, pe0)
      for e in range(NREG):
        d = pl.ds(e * NLANE, NLANE)
        outb[SEQ - 1, d] = vrows[SEQ - 1, d] + pe_last[e] + eosv[e]

      pltpu.sync_copy(outb, out_hbm.at[pl.ds(seq * SEQ, SEQ)])
      return 0

    lax.fori_loop(0, seq_per_worker, seq_body, 0)

  return body


def kernel(value, depth, position, value_table, spatial_table_0,
           spatial_table_1, spatial_table_2, eos):
  del depth
  n = value.shape[0]
  n_workers = 32
  seq_per_worker = n // n_workers

  val_idx = value.astype(jnp.int32).reshape(n * 2, HALF)
  pos_idx = (position.astype(jnp.int32)
             + jnp.array([0, 512, 1024], jnp.int32)).reshape(n * 6, HALF)
  spatial_cat = jnp.concatenate(
      [spatial_table_0, spatial_table_1, spatial_table_2], axis=0)

  out = _sc_kernel(n, seq_per_worker)(
      val_idx, pos_idx, value_table, spatial_cat, eos)
  return out.reshape(n, SEQ, EMBED)


# speedup vs baseline: 6.0424x; 1.5363x over previous
import functools

import jax
import jax.numpy as jnp
from jax import lax
from jax.experimental import pallas as pl
from jax.experimental.pallas import tpu as pltpu
from jax.experimental.pallas import tpu_sc as plsc

EMBED = 64
SEQ = 200
U = 100
NLANE = 16
NREG = EMBED // NLANE
NWORKERS = 32


def _sc_kernel(n_seqs):
  units_per_worker = (n_seqs // NWORKERS) * 2
  mesh = plsc.VectorSubcoreMesh(
      core_axis_name="c", subcore_axis_name="s", num_cores=2, num_subcores=16)

  def pair(ty):
    return [ty() for _ in range(2)]

  @functools.partial(
      pl.kernel,
      out_type=jax.ShapeDtypeStruct((n_seqs * SEQ, EMBED), jnp.float32),
      mesh=mesh,
      compiler_params=pltpu.CompilerParams(use_tc_tiling_on_sc=False),
      scratch_types=[
          pair(lambda: pltpu.VMEM((U,), jnp.int32)),
          pair(lambda: [pltpu.VMEM((U,), jnp.int32) for _ in range(3)]),
          pair(lambda: [pltpu.VMEM((U,), jnp.int32) for _ in range(3)]),
          pair(lambda: pltpu.VMEM((U, EMBED), jnp.float32)),
          pair(lambda: pltpu.VMEM((3 * U, EMBED), jnp.float32)),
          pair(lambda: pltpu.VMEM((U, EMBED), jnp.float32)),
          pltpu.VMEM((EMBED,), jnp.float32),
          pair(lambda: pltpu.SemaphoreType.DMA),
          pair(lambda: pltpu.SemaphoreType.DMA),
          pair(lambda: pltpu.SemaphoreType.DMA),
      ],
  )
  def body(val_hbm, pos_hbm, vtab_hbm, stab_hbm, eos_hbm, out_hbm,
           vidx, praw, poff, vrows, srows, outb, eosb,
           sem_i, sem_g, sem_o):
    cid = lax.axis_index("c")
    sid = lax.axis_index("s")
    wid = sid * 2 + cid
    u0 = wid * units_per_worker

    pltpu.sync_copy(eos_hbm, eosb)
    eosv = [eosb[pl.ds(e * NLANE, NLANE)] for e in range(NREG)]
    iota = lax.iota(jnp.int32, NLANE)
    offs = [((iota + p) % 3) * 512 for p in range(3)]

    def issue_idx(g, s):
      pltpu.async_copy(val_hbm.at[g], vidx[s], sem_i[s])
      for r in range(3):
        pltpu.async_copy(pos_hbm.at[3 * g + r], praw[s][r], sem_i[s])

    def wait_idx(s):
      pltpu.make_async_copy(val_hbm.at[0], vidx[s], sem_i[s]).wait()
      for r in range(3):
        pltpu.make_async_copy(pos_hbm.at[0], praw[s][r], sem_i[s]).wait()

    def offadd(s):
      for r in range(3):
        for k in range(6):
          d = pl.ds(k * NLANE, NLANE)
          poff[s][r][d] = praw[s][r][d] + offs[(r + k) % 3]
        d = pl.ds(U - NLANE, NLANE)
        poff[s][r][d] = praw[s][r][d] + offs[r % 3]

    def issue_gathers(s):
      pltpu.async_copy(vtab_hbm.at[vidx[s]], vrows[s], sem_g[s])
      for r in range(3):
        pltpu.async_copy(stab_hbm.at[poff[s][r]],
                         srows[s].at[pl.ds(r * U, U)], sem_g[s])

    def wait_gathers(s):
      pltpu.make_async_copy(vtab_hbm.at[vidx[s]], vrows[s], sem_g[s]).wait()
      for r in range(3):
        pltpu.make_async_copy(stab_hbm.at[poff[s][r]],
                              srows[s].at[pl.ds(r * U, U)], sem_g[s]).wait()

    def start_out(g, s):
      pltpu.async_copy(outb[s], out_hbm.at[pl.ds(g * U, U)], sem_o[s])

    def wait_out(s):
      pltpu.make_async_copy(outb[s], out_hbm.at[pl.ds(0, U)], sem_o[s]).wait()

    def pe_at(s, r0):
      return [srows[s][r0, pl.ds(e * NLANE, NLANE)]
              + srows[s][r0 + 1, pl.ds(e * NLANE, NLANE)]
              + srows[s][r0 + 2, pl.ds(e * NLANE, NLANE)]
              for e in range(NREG)]

    def emit_token(s, t, pe):
      nxt = pe_at(s, 3 * t + 3)
      for e in range(NREG):
        d = pl.ds(e * NLANE, NLANE)
        outb[s][t, d] = vrows[s][t, d] + pe[e] + nxt[e]
      return nxt

    def compute_unit(s):
      pe = pe_at(s, 0)

      def tok4(i, pe):
        pe = list(pe)
        for q in range(4):
          pe = emit_token(s, 4 * i + q, pe)
        return tuple(pe)

      pe = list(lax.fori_loop(0, 24, tok4, tuple(pe)))
      for t in range(96, 99):
        pe = emit_token(s, t, pe)
      return pe

    issue_idx(u0, 0)
    issue_idx(u0 + 1, 1)
    wait_idx(0)
    offadd(0)
    issue_gathers(0)

    def step(it, _):
      even = u0 + 2 * it

      wait_idx(1)
      offadd(1)
      wait_gathers(0)
      issue_gathers(1)

      @pl.when(it <= units_per_worker // 2 - 2)
      def _():
        issue_idx(even + 2, 0)

      @pl.when(it >= 1)
      def _():
        wait_out(0)
      pe_carry = compute_unit(0)

      wait_gathers(1)
      nxt = pe_at(1, 0)
      for e in range(NREG):
        d = pl.ds(e * NLANE, NLANE)
        outb[0][U - 1, d] = vrows[0][U - 1, d] + pe_carry[e] + nxt[e]
      start_out(even, 0)

      @pl.when(it <= units_per_worker // 2 - 2)
      def _():
        wait_idx(0)
        offadd(0)
        issue_gathers(0)
        issue_idx(even + 3, 1)

      @pl.when(it >= 1)
      def _():
        wait_out(1)
      pe_last = compute_unit(1)
      for e in range(NREG):
        d = pl.ds(e * NLANE, NLANE)
        outb[1][U - 1, d] = vrows[1][U - 1, d] + pe_last[e] + eosv[e]
      start_out(even + 1, 1)
      return 0

    lax.fori_loop(0, units_per_worker // 2, step, 0)
    wait_out(0)
    wait_out(1)

  return body


def kernel(value, depth, position, value_table, spatial_table_0,
           spatial_table_1, spatial_table_2, eos):
  del depth
  n = value.shape[0]
  val2 = value.astype(jnp.int32).reshape(n * 2, U)
  pos2 = position.astype(jnp.int32).reshape(n * 6, U)
  stab = jnp.concatenate(
      [spatial_table_0, spatial_table_1, spatial_table_2], axis=0)
  out = _sc_kernel(n)(val2, pos2, value_table, stab, eos)
  return out.reshape(n, SEQ, EMBED)


# speedup vs baseline: 6.1329x; 1.0150x over previous
import functools

import jax
import jax.numpy as jnp
from jax import lax
from jax.experimental import pallas as pl
from jax.experimental.pallas import tpu as pltpu
from jax.experimental.pallas import tpu_sc as plsc

EMBED = 64
SEQ = 200
NB = 128
NLANE = 16
NREG = EMBED // NLANE
NWORKERS = 32
NBLOCKS = 8
SRANGES = NWORKERS // NBLOCKS
SSTEP = SEQ // SRANGES


def _sc_kernel(n_seqs):
  assert n_seqs == NBLOCKS * NB
  mesh = plsc.VectorSubcoreMesh(
      core_axis_name="c", subcore_axis_name="s", num_cores=2, num_subcores=16)

  def pair(ty):
    return [ty() for _ in range(2)]

  @functools.partial(
      pl.kernel,
      out_type=jax.ShapeDtypeStruct((SEQ, NREG * 2, NBLOCKS, 8, NB),
                                    jnp.float32),
      mesh=mesh,
      compiler_params=pltpu.CompilerParams(
          use_tc_tiling_on_sc=False, needs_layout_passes=False),
      scratch_types=[
          pair(lambda: pltpu.VMEM((NB,), jnp.int32)),
          pair(lambda: [pltpu.VMEM((NB,), jnp.int32) for _ in range(3)]),
          pair(lambda: pltpu.VMEM((NB, EMBED), jnp.float32)),
          pair(lambda: [pltpu.VMEM((NB, EMBED), jnp.float32) for _ in range(3)]),
          pair(lambda: pltpu.VMEM((NREG * 2, 8, NB), jnp.float32)),
          pair(lambda: pltpu.VMEM((NB, EMBED), jnp.float32)),
          pltpu.VMEM((EMBED,), jnp.float32),
          pair(lambda: pltpu.SemaphoreType.DMA),
          pair(lambda: pltpu.SemaphoreType.DMA),
          pair(lambda: pltpu.SemaphoreType.DMA),
      ],
  )
  def body(valT, posT, vtab, st0, st1, st2, eos_hbm, out_hbm,
           vidx, pidx, vrows, srows, outb, peb, eosb,
           sem_i, sem_g, sem_o):
    stabs = [st0, st1, st2]
    cid = lax.axis_index("c")
    sid = lax.axis_index("s")
    wid = sid * 2 + cid
    nt = wid % NBLOCKS
    s0 = (wid // NBLOCKS) * SSTEP
    n0 = nt * NB

    pltpu.sync_copy(eos_hbm, eosb)

    def issue_idx(j, X):
      s = s0 + j
      pltpu.async_copy(valT.at[s, pl.ds(n0, NB)], vidx[X], sem_i[X])
      sp = jnp.minimum(s + 1, SEQ - 1)
      for a in range(3):
        pltpu.async_copy(posT.at[a * SEQ + sp, pl.ds(n0, NB)],
                         pidx[X][a], sem_i[X])

    def wait_idx(X):
      pltpu.make_async_copy(valT.at[0, pl.ds(0, NB)], vidx[X],
                            sem_i[X]).wait()
      for a in range(3):
        pltpu.make_async_copy(posT.at[0, pl.ds(0, NB)], pidx[X][a],
                              sem_i[X]).wait()

    def issue_gathers(X):
      pltpu.async_copy(vtab.at[vidx[X]], vrows[X], sem_g[X])
      for a in range(3):
        pltpu.async_copy(stabs[a].at[pidx[X][a]], srows[X][a], sem_g[X])

    def wait_gathers(X):
      pltpu.make_async_copy(vtab.at[vidx[X]], vrows[X], sem_g[X]).wait()
      for a in range(3):
        pltpu.make_async_copy(stabs[a].at[pidx[X][a]], srows[X][a],
                              sem_g[X]).wait()

    def start_out(j, X):
      s = s0 + j
      pltpu.async_copy(outb[X], out_hbm.at[s, :, nt, :, :], sem_o[X])

    def wait_out(X):
      pltpu.make_async_copy(outb[X], out_hbm.at[0, :, 0, :, :], sem_o[X]).wait()

    iota = lax.iota(jnp.int32, NLANE)
    et_q = [lax.shift_right_logical(iota + q * NLANE, 3) for q in range(NREG)]
    ei_q = [lax.bitwise_and(iota + q * NLANE, 7) for q in range(NREG)]
    eosv = [eosb[pl.ds(q * NLANE, NLANE)] for q in range(NREG)]

    def spatial_sum(X, n, q):
      d = pl.ds(q * NLANE, NLANE)
      return (srows[X][0][n, d] + srows[X][1][n, d] + srows[X][2][n, d])

    def compute_step(j, X, cur, nxt):
      def tok(n):
        nv = lax.broadcast(n, (NLANE,))
        for q in range(NREG):
          d = pl.ds(q * NLANE, NLANE)
          pn = spatial_sum(X, n, q)
          peb[nxt][n, d] = pn
          ov = vrows[X][n, d] + peb[cur][n, d] + pn
          plsc.store_scatter(outb[X], [et_q[q], ei_q[q], nv], ov)

      def tok4(i, _):
        for u in range(4):
          tok(4 * i + u)
        return 0

      lax.fori_loop(0, NB // 4, tok4, 0)

      @pl.when(s0 + j == SEQ - 1)
      def _():
        def fix4(i, _):
          for u in range(4):
            n = 4 * i + u
            nv = lax.broadcast(n, (NLANE,))
            for q in range(NREG):
              d = pl.ds(q * NLANE, NLANE)
              ov = vrows[X][n, d] + peb[cur][n, d] + eosv[q]
              plsc.store_scatter(outb[X], [et_q[q], ei_q[q], nv], ov)
          return 0
        lax.fori_loop(0, NB // 4, fix4, 0)

    for a in range(3):
      pltpu.async_copy(posT.at[a * SEQ + s0, pl.ds(n0, NB)],
                       pidx[1][a], sem_i[1])
    issue_idx(0, 0)
    for a in range(3):
      pltpu.make_async_copy(posT.at[0, pl.ds(0, NB)], pidx[1][a],
                            sem_i[1]).wait()
    for a in range(3):
      pltpu.async_copy(stabs[a].at[pidx[1][a]], srows[1][a], sem_g[1])
    wait_idx(0)
    issue_gathers(0)
    for a in range(3):
      pltpu.make_async_copy(stabs[a].at[pidx[1][a]], srows[1][a],
                            sem_g[1]).wait()

    def pe0_4(i, _):
      for u in range(4):
        n = 4 * i + u
        for q in range(NREG):
          peb[0][n, pl.ds(q * NLANE, NLANE)] = spatial_sum(1, n, q)
      return 0
    lax.fori_loop(0, NB // 4, pe0_4, 0)
    issue_idx(1, 1)

    npairs = SSTEP // 2

    def step_pair(i, _):
      wait_idx(1)
      wait_gathers(0)
      issue_gathers(1)

      @pl.when(i <= npairs - 2)
      def _():
        issue_idx(2 * i + 2, 0)

      @pl.when(i >= 1)
      def _():
        wait_out(0)
      compute_step(2 * i, 0, 0, 1)
      start_out(2 * i, 0)

      wait_gathers(1)

      @pl.when(i <= npairs - 2)
      def _():
        wait_idx(0)
        issue_gathers(0)
        issue_idx(2 * i + 3, 1)

      @pl.when(i >= 1)
      def _():
        wait_out(1)
      compute_step(2 * i + 1, 1, 1, 0)
      start_out(2 * i + 1, 1)
      return 0

    lax.fori_loop(0, npairs, step_pair, 0)
    wait_out(0)
    wait_out(1)

  return body


def kernel(value, depth, position, value_table, spatial_table_0,
           spatial_table_1, spatial_table_2, eos):
  del depth
  n = value.shape[0]
  valT = value.astype(jnp.int32).T
  posT = position.astype(jnp.int32).transpose(2, 1, 0).reshape(3 * SEQ, n)
  out = _sc_kernel(n)(valT, posT, value_table,
                      spatial_table_0, spatial_table_1, spatial_table_2, eos)
  out = out.transpose(2, 4, 0, 1, 3).reshape(n, SEQ, EMBED)
  return out


# speedup vs baseline: 9.6133x; 1.5675x over previous
import functools

import jax
import jax.numpy as jnp
from jax import lax
from jax.experimental import pallas as pl
from jax.experimental.pallas import tpu as pltpu
from jax.experimental.pallas import tpu_sc as plsc

EMBED = 64
SEQ = 200
NB = 128
NLANE = 16
NREG = EMBED // NLANE
NWORKERS = 32
NBLOCKS = 8
SRANGES = NWORKERS // NBLOCKS
SSTEP = SEQ // SRANGES


def _sc_kernel(n_seqs):
  assert n_seqs == NBLOCKS * NB
  mesh = plsc.VectorSubcoreMesh(
      core_axis_name="c", subcore_axis_name="s", num_cores=2, num_subcores=16)

  def pair(ty):
    return [ty() for _ in range(2)]

  @functools.partial(
      pl.kernel,
      out_type=jax.ShapeDtypeStruct((SEQ, NREG * 2, NBLOCKS, 8, NB),
                                    jnp.float32),
      mesh=mesh,
      compiler_params=pltpu.CompilerParams(
          use_tc_tiling_on_sc=False, needs_layout_passes=False),
      scratch_types=[
          pair(lambda: pltpu.VMEM((NB,), jnp.int32)),
          pair(lambda: [pltpu.VMEM((NB,), jnp.int32) for _ in range(3)]),
          pair(lambda: pltpu.VMEM((NB, EMBED), jnp.float32)),
          pair(lambda: [pltpu.VMEM((NB, EMBED), jnp.float32) for _ in range(3)]),
          pair(lambda: pltpu.VMEM((NREG * 2, 8, NB + 1), jnp.float32)),
          pair(lambda: pltpu.VMEM((NB, EMBED), jnp.float32)),
          pltpu.VMEM((EMBED,), jnp.float32),
          pair(lambda: pltpu.SemaphoreType.DMA),
          pair(lambda: pltpu.SemaphoreType.DMA),
          pair(lambda: pltpu.SemaphoreType.DMA),
      ],
  )
  def body(valT, posT, vtab, st0, st1, st2, eos_hbm, out_hbm,
           vidx, pidx, vrows, srows, outb, peb, eosb,
           sem_i, sem_g, sem_o):
    stabs = [st0, st1, st2]
    cid = lax.axis_index("c")
    sid = lax.axis_index("s")
    wid = sid * 2 + cid
    nt = wid % NBLOCKS
    s0 = (wid // NBLOCKS) * SSTEP
    n0 = nt * NB

    pltpu.sync_copy(eos_hbm, eosb)

    def issue_idx(j, X):
      s = s0 + j
      pltpu.async_copy(valT.at[s, pl.ds(n0, NB)], vidx[X], sem_i[X])
      sp = jnp.minimum(s + 1, SEQ - 1)
      for a in range(3):
        pltpu.async_copy(posT.at[a * SEQ + sp, pl.ds(n0, NB)],
                         pidx[X][a], sem_i[X])

    def wait_idx(X):
      pltpu.make_async_copy(valT.at[0, pl.ds(0, NB)], vidx[X],
                            sem_i[X]).wait()
      for a in range(3):
        pltpu.make_async_copy(posT.at[0, pl.ds(0, NB)], pidx[X][a],
                              sem_i[X]).wait()

    def issue_gathers(X):
      pltpu.async_copy(vtab.at[vidx[X]], vrows[X], sem_g[X])
      for a in range(3):
        pltpu.async_copy(stabs[a].at[pidx[X][a]], srows[X][a], sem_g[X])

    def wait_gathers(X):
      pltpu.make_async_copy(vtab.at[vidx[X]], vrows[X], sem_g[X]).wait()
      for a in range(3):
        pltpu.make_async_copy(stabs[a].at[pidx[X][a]], srows[X][a],
                              sem_g[X]).wait()

    def start_out(j, X):
      s = s0 + j
      pltpu.async_copy(outb[X].at[:, :, pl.ds(0, NB)],
                       out_hbm.at[s, :, nt, :, :], sem_o[X])

    def wait_out(X):
      pltpu.make_async_copy(outb[X].at[:, :, pl.ds(0, NB)],
                            out_hbm.at[0, :, 0, :, :], sem_o[X]).wait()

    iota = lax.iota(jnp.int32, NLANE)
    et_q = [lax.shift_right_logical(iota + q * NLANE, 3) for q in range(NREG)]
    ei_q = [lax.bitwise_and(iota + q * NLANE, 7) for q in range(NREG)]
    eosv = [eosb[pl.ds(q * NLANE, NLANE)] for q in range(NREG)]

    def spatial_sum(X, n, q):
      d = pl.ds(q * NLANE, NLANE)
      return (srows[X][0][n, d] + srows[X][1][n, d] + srows[X][2][n, d])

    def compute_step(j, X, cur, nxt):
      def tok(n):
        nv = lax.broadcast(n, (NLANE,))
        for q in range(NREG):
          d = pl.ds(q * NLANE, NLANE)
          pn = spatial_sum(X, n, q)
          peb[nxt][n, d] = pn
          ov = vrows[X][n, d] + peb[cur][n, d] + pn
          plsc.store_scatter(outb[X], [et_q[q], ei_q[q], nv], ov)

      def tok4(i, _):
        for u in range(4):
          tok(4 * i + u)
        return 0

      lax.fori_loop(0, NB // 4, tok4, 0)

      @pl.when(s0 + j == SEQ - 1)
      def _():
        def fix4(i, _):
          for u in range(4):
            n = 4 * i + u
            nv = lax.broadcast(n, (NLANE,))
            for q in range(NREG):
              d = pl.ds(q * NLANE, NLANE)
              ov = vrows[X][n, d] + peb[cur][n, d] + eosv[q]
              plsc.store_scatter(outb[X], [et_q[q], ei_q[q], nv], ov)
          return 0
        lax.fori_loop(0, NB // 4, fix4, 0)

    for a in range(3):
      pltpu.async_copy(posT.at[a * SEQ + s0, pl.ds(n0, NB)],
                       pidx[1][a], sem_i[1])
    issue_idx(0, 0)
    for a in range(3):
      pltpu.make_async_copy(posT.at[0, pl.ds(0, NB)], pidx[1][a],
                            sem_i[1]).wait()
    for a in range(3):
      pltpu.async_copy(stabs[a].at[pidx[1][a]], srows[1][a], sem_g[1])
    wait_idx(0)
    issue_gathers(0)
    for a in range(3):
      pltpu.make_async_copy(stabs[a].at[pidx[1][a]], srows[1][a],
                            sem_g[1]).wait()

    def pe0_4(i, _):
      for u in range(4):
        n = 4 * i + u
        for q in range(NREG):
          peb[0][n, pl.ds(q * NLANE, NLANE)] = spatial_sum(1, n, q)
      return 0
    lax.fori_loop(0, NB // 4, pe0_4, 0)
    issue_idx(1, 1)

    npairs = SSTEP // 2

    def step_pair(i, _):
      wait_idx(1)
      wait_gathers(0)
      issue_gathers(1)

      @pl.when(i <= npairs - 2)
      def _():
        issue_idx(2 * i + 2, 0)

      @pl.when(i >= 1)
      def _():
        wait_out(0)
      compute_step(2 * i, 0, 0, 1)
      start_out(2 * i, 0)

      wait_gathers(1)

      @pl.when(i <= npairs - 2)
      def _():
        wait_idx(0)
        issue_gathers(0)
        issue_idx(2 * i + 3, 1)

      @pl.when(i >= 1)
      def _():
        wait_out(1)
      compute_step(2 * i + 1, 1, 1, 0)
      start_out(2 * i + 1, 1)
      return 0

    lax.fori_loop(0, npairs, step_pair, 0)
    wait_out(0)
    wait_out(1)

  return body


def kernel(value, depth, position, value_table, spatial_table_0,
           spatial_table_1, spatial_table_2, eos):
  del depth
  n = value.shape[0]
  valT = value.astype(jnp.int32).T
  posT = position.astype(jnp.int32).transpose(2, 1, 0).reshape(3 * SEQ, n)
  out = _sc_kernel(n)(valT, posT, value_table,
                      spatial_table_0, spatial_table_1, spatial_table_2, eos)
  out = out.transpose(2, 4, 0, 1, 3).reshape(n, SEQ, EMBED)
  return out


# speedup vs baseline: 10.2117x; 1.0623x over previous
import functools

import jax
import jax.numpy as jnp
import numpy as np
from jax import lax
from jax.experimental import pallas as pl
from jax.experimental.pallas import tpu as pltpu
from jax.experimental.pallas import tpu_sc as plsc

EMBED = 64
SEQ = 200
NB = 128
NLANE = 16
NREG = EMBED // NLANE
NWORKERS = 32
NBLOCKS = 8
SRANGES = NWORKERS // NBLOCKS
SSTEP = SEQ // SRANGES
NTAB = 3 * 512

_PERM = np.empty((EMBED,), np.int64)
for _k in range(2):
  for _i in range(NLANE):
    _PERM[32 * _k + 2 * _i] = 32 * _k + _i
    _PERM[32 * _k + 2 * _i + 1] = 32 * _k + NLANE + _i


def _sc_kernel(n_seqs):
  assert n_seqs == NBLOCKS * NB
  mesh = plsc.VectorSubcoreMesh(
      core_axis_name="c", subcore_axis_name="s", num_cores=2, num_subcores=16)

  def pair(ty):
    return [ty() for _ in range(2)]

  @functools.partial(
      pl.kernel,
      out_type=jax.ShapeDtypeStruct((SEQ, NREG * 2, NBLOCKS, 8, NB),
                                    jnp.float32),
      mesh=mesh,
      compiler_params=pltpu.CompilerParams(
          use_tc_tiling_on_sc=False, needs_layout_passes=False),
      scratch_types=[
          pair(lambda: pltpu.VMEM((NB,), jnp.int32)),
          pair(lambda: [pltpu.VMEM((NB,), jnp.int32) for _ in range(3)]),
          pair(lambda: pltpu.VMEM((NB, EMBED), jnp.float32)),
          pltpu.VMEM((NTAB * EMBED,), jnp.bfloat16),
          pair(lambda: pltpu.VMEM((NREG * 2, 8, NB + 1), jnp.float32)),
          pair(lambda: pltpu.VMEM((NB, EMBED), jnp.float32)),
          pltpu.VMEM((3, NB), jnp.int32),
          pltpu.VMEM((EMBED,), jnp.float32),
          pair(lambda: pltpu.SemaphoreType.DMA),
          pair(lambda: pltpu.SemaphoreType.DMA),
          pair(lambda: pltpu.SemaphoreType.DMA),
      ],
  )
  def body(valT, posT, vtab, stab, eos_hbm, out_hbm,
           vidx, pidx, vrows, tabv, outb, peb, hold, eosb,
           sem_i, sem_g, sem_o):
    cid = lax.axis_index("c")
    sid = lax.axis_index("s")
    wid = sid * 2 + cid
    nt = wid % NBLOCKS
    s0 = (wid // NBLOCKS) * SSTEP
    n0 = nt * NB

    pltpu.sync_copy(eos_hbm, eosb)
    pltpu.sync_copy(stab, tabv)

    def issue_idx(j, X):
      s = s0 + j
      pltpu.async_copy(valT.at[s, pl.ds(n0, NB)], vidx[X], sem_i[X])
      sp = jnp.minimum(s + 1, SEQ - 1)
      for a in range(3):
        pltpu.async_copy(posT.at[a * SEQ + sp, pl.ds(n0, NB)],
                         pidx[X][a], sem_i[X])

    def wait_idx(X):
      pltpu.make_async_copy(valT.at[0, pl.ds(0, NB)], vidx[X],
                            sem_i[X]).wait()
      for a in range(3):
        pltpu.make_async_copy(posT.at[0, pl.ds(0, NB)], pidx[X][a],
                              sem_i[X]).wait()

    def issue_gathers(X):
      pltpu.async_copy(vtab.at[vidx[X]], vrows[X], sem_g[X])

    def wait_gathers(X):
      pltpu.make_async_copy(vtab.at[vidx[X]], vrows[X], sem_g[X]).wait()

    def start_out(j, X):
      s = s0 + j
      pltpu.async_copy(outb[X].at[:, :, pl.ds(0, NB)],
                       out_hbm.at[s, :, nt, :, :], sem_o[X])

    def wait_out(X):
      pltpu.make_async_copy(outb[X].at[:, :, pl.ds(0, NB)],
                            out_hbm.at[0, :, 0, :, :], sem_o[X]).wait()

    iota = lax.iota(jnp.int32, NLANE)
    et_q = [lax.shift_right_logical(iota + q * NLANE, 3) for q in range(NREG)]
    ei_q = [lax.bitwise_and(iota + q * NLANE, 7) for q in range(NREG)]
    eosv = [eosb[pl.ds(q * NLANE, NLANE)] for q in range(NREG)]

    def pe_quads(ps):
      qs = [None] * NREG
      for a in range(3):
        p = ps[a]
        for h in range(2):
          lo, hi = plsc.unpack(tabv[pl.ds(p * EMBED + 32 * h, 32)],
                               format=plsc.PackFormat.INTERLEAVED,
                               preferred_element_type=jnp.float32)
          for j, u in ((2 * h, lo), (2 * h + 1, hi)):
            qs[j] = u if qs[j] is None else qs[j] + u
      return qs

    def snap_idx(X):
      for a in range(3):
        for kk in range(NB // NLANE):
          d = pl.ds(kk * NLANE, NLANE)
          hold[a, d] = pidx[X][a][d] + a * 512

    def idx_vecs(g):
      return [hold[a, pl.ds(g * NLANE, NLANE)] for a in range(3)]

    def compute_step(j, X, cur, nxt):
      def grp(g, _):
        pv = idx_vecs(g)
        for u in range(NLANE):
          n = g * NLANE + u
          nv = lax.broadcast(n, (NLANE,))
          pn = pe_quads([pv[a][u] for a in range(3)])
          for q in range(NREG):
            d = pl.ds(q * NLANE, NLANE)
            peb[nxt][n, d] = pn[q]
            ov = vrows[X][n, d] + peb[cur][n, d] + pn[q]
            plsc.store_scatter(outb[X], [et_q[q], ei_q[q], nv], ov)
        return 0

      lax.fori_loop(0, NB // NLANE, grp, 0)

      @pl.when(s0 + j == SEQ - 1)
      def _():
        def fix4(i, _):
          for u in range(4):
            n = 4 * i + u
            nv = lax.broadcast(n, (NLANE,))
            for q in range(NREG):
              d = pl.ds(q * NLANE, NLANE)
              ov = vrows[X][n, d] + peb[cur][n, d] + eosv[q]
              plsc.store_scatter(outb[X], [et_q[q], ei_q[q], nv], ov)
          return 0
        lax.fori_loop(0, NB // 4, fix4, 0)

    for a in range(3):
      pltpu.async_copy(posT.at[a * SEQ + s0, pl.ds(n0, NB)],
                       pidx[1][a], sem_i[1])
    issue_idx(0, 0)
    for a in range(3):
      pltpu.make_async_copy(posT.at[0, pl.ds(0, NB)], pidx[1][a],
                            sem_i[1]).wait()
    wait_idx(0)
    issue_gathers(0)

    def pe0_grp(g, _):
      pv = idx_vecs(g)
      for u in range(NLANE):
        n = g * NLANE + u
        pn = pe_quads([pv[a][u] for a in range(3)])
        for q in range(NREG):
          peb[0][n, pl.ds(q * NLANE, NLANE)] = pn[q]
      return 0
    snap_idx(1)
    lax.fori_loop(0, NB // NLANE, pe0_grp, 0)
    issue_idx(1, 1)

    npairs = SSTEP // 2

    def step_pair(i, _):
      wait_idx(1)
      wait_gathers(0)
      snap_idx(0)

      @pl.when(i <= npairs - 2)
      def _():
        issue_idx(2 * i + 2, 0)
      issue_gathers(1)

      @pl.when(i >= 1)
      def _():
        wait_out(0)
      compute_step(2 * i, 0, 0, 1)
      start_out(2 * i, 0)

      wait_gathers(1)
      snap_idx(1)

      @pl.when(i <= npairs - 2)
      def _():
        wait_idx(0)
        issue_gathers(0)
        issue_idx(2 * i + 3, 1)

      @pl.when(i >= 1)
      def _():
        wait_out(1)
      compute_step(2 * i + 1, 1, 1, 0)
      start_out(2 * i + 1, 1)
      return 0

    lax.fori_loop(0, npairs, step_pair, 0)
    wait_out(0)
    wait_out(1)

  return body


def kernel(value, depth, position, value_table, spatial_table_0,
           spatial_table_1, spatial_table_2, eos):
  del depth
  n = value.shape[0]
  valT = value.astype(jnp.int32).T
  posT = position.astype(jnp.int32).transpose(2, 1, 0).reshape(3 * SEQ, n)
  stab = jnp.concatenate(
      [spatial_table_0, spatial_table_1, spatial_table_2], axis=0)
  stab = stab[:, _PERM].astype(jnp.bfloat16).reshape(-1)
  out = _sc_kernel(n)(valT, posT, value_table, stab, eos)
  out = out.transpose(2, 4, 0, 1, 3).reshape(n, SEQ, EMBED)
  return out
